# baseline (speedup 1.0000x reference)
import jax
import jax.numpy as jnp
from jax import lax
from jax.experimental import pallas as pl
from jax.experimental.pallas import tpu as pltpu
from jax.experimental.pallas import tpu_sc as plsc

N_NODES = 10000
N_EDGES = 320000
D_FEAT = 128

NC = 2
NS = 16
NW = NC * NS
CHUNK = 80
N_CHUNKS = 128
E_PAD = NW * N_CHUNKS * CHUNK
N_PAD = 10112
ROWS_PER_TILE = N_PAD // NS


def _sc_body(src_hbm, dst_hbm, x_hbm, zacc_hbm, zdeg_hbm, ones_hbm,
             part_out, deg_out,
             acc, deg, src_all, dst_all, rows_v, ones_v, dtmp, sem):
    cid = lax.axis_index("c")
    sid = lax.axis_index("s")
    wid = cid * NS + sid

    r0 = sid * ROWS_PER_TILE
    pltpu.sync_copy(zacc_hbm, acc.at[pl.ds(r0, ROWS_PER_TILE), :])
    pltpu.sync_copy(zdeg_hbm, dtmp)
    pltpu.sync_copy(dtmp, deg.at[pl.ds(r0, ROWS_PER_TILE)])
    pltpu.sync_copy(ones_hbm, ones_v)
    pltpu.sync_copy(src_hbm.at[wid], src_all)
    pltpu.sync_copy(dst_hbm.at[wid], dst_all)
    plsc.subcore_barrier()

    def chunk_body(j, carry):
        pltpu.async_copy(x_hbm.at[src_all.at[j]], rows_v, sem).wait()
        pltpu.sync_copy(rows_v, acc.at[dst_all.at[j]], add=True)
        pltpu.sync_copy(ones_v, deg.at[dst_all.at[j]], add=True)
        return carry

    lax.fori_loop(0, N_CHUNKS, chunk_body, 0)
    plsc.subcore_barrier()

    pltpu.sync_copy(acc.at[pl.ds(r0, ROWS_PER_TILE), :],
                    part_out.at[cid, pl.ds(r0, ROWS_PER_TILE), :])
    pltpu.sync_copy(deg.at[pl.ds(r0, ROWS_PER_TILE)], dtmp)
    pltpu.sync_copy(dtmp, deg_out.at[pl.ds(cid * N_PAD + r0, ROWS_PER_TILE)])


_sc_aggregate = pl.kernel(
    _sc_body,
    out_type=(
        jax.ShapeDtypeStruct((NC, N_PAD, D_FEAT), jnp.float32),
        jax.ShapeDtypeStruct((NC * N_PAD,), jnp.float32),
    ),
    mesh=plsc.VectorSubcoreMesh(core_axis_name="c", subcore_axis_name="s",
                                num_cores=NC, num_subcores=NS),
    scratch_types=[
        pltpu.VMEM_SHARED((N_PAD, D_FEAT), jnp.float32),
        pltpu.VMEM_SHARED((N_PAD,), jnp.float32),
        pltpu.VMEM((N_CHUNKS, CHUNK), jnp.int32),
        pltpu.VMEM((N_CHUNKS, CHUNK), jnp.int32),
        pltpu.VMEM((CHUNK, D_FEAT), jnp.float32),
        pltpu.VMEM((CHUNK,), jnp.float32),
        pltpu.VMEM((ROWS_PER_TILE,), jnp.float32),
        pltpu.SemaphoreType.DMA,
    ],
)


def _combine_body(p_ref, d_ref, o_ref):
    s = p_ref[0] + p_ref[1]
    d = d_ref[0] + d_ref[1]
    out = jnp.where(d > 0.0, s / d, 0.0)
    o_ref[...] = out[:N_NODES, :]


def _combine(part, degp):
    return pl.pallas_call(
        _combine_body,
        in_specs=[
            pl.BlockSpec((NC, N_PAD, D_FEAT), lambda: (0, 0, 0)),
            pl.BlockSpec((NC, N_PAD, 1), lambda: (0, 0, 0)),
        ],
        out_specs=pl.BlockSpec((N_NODES, D_FEAT), lambda: (0, 0)),
        out_shape=jax.ShapeDtypeStruct((N_NODES, D_FEAT), jnp.float32),
    )(part, degp)


@jax.jit
def kernel(input, adj):
    pad = E_PAD - N_EDGES
    dst = jnp.concatenate([adj[0], jnp.full((pad,), N_NODES, jnp.int32)])
    src = jnp.concatenate([adj[1], jnp.zeros((pad,), jnp.int32)])
    dst = dst.reshape(NW, N_CHUNKS, CHUNK)
    src = src.reshape(NW, N_CHUNKS, CHUNK)
    zacc = jnp.zeros((ROWS_PER_TILE, D_FEAT), jnp.float32)
    zdeg = jnp.zeros((ROWS_PER_TILE,), jnp.float32)
    ones = jnp.ones((CHUNK,), jnp.float32)
    part, degflat = _sc_aggregate(src, dst, input, zacc, zdeg, ones)
    return _combine(part, degflat.reshape(NC, N_PAD, 1))


# speedup vs baseline: 2.5730x; 2.5730x over previous
import jax
import jax.numpy as jnp
from jax import lax
from jax.experimental import pallas as pl
from jax.experimental.pallas import tpu as pltpu
from jax.experimental.pallas import tpu_sc as plsc

N_NODES = 10000
N_EDGES = 320000
D_FEAT = 128

NC = 2
NS = 16
NW = NC * NS
E_PER_TILE = N_EDGES // NW
CHUNK = 80
N_CHUNKS = E_PER_TILE // CHUNK
N_PAIR = (N_CHUNKS - 1) // 2
N_PAD = 10112
ROWS_PER_TILE = N_PAD // NS


def _sc_body(src_hbm, dst_hbm, x_hbm, zacc_hbm, zdeg_hbm, ones_hbm,
             part_out, deg_out,
             acc, deg, src_a, dst_a, src_b, dst_b, rows_a, rows_b,
             ones_v, dtmp, sem_a, sem_b):
    cid = lax.axis_index("c")
    sid = lax.axis_index("s")
    wid = cid * NS + sid

    r0 = sid * ROWS_PER_TILE
    pltpu.sync_copy(zacc_hbm, acc.at[pl.ds(r0, ROWS_PER_TILE), :])
    pltpu.sync_copy(zdeg_hbm, dtmp)
    pltpu.sync_copy(dtmp, deg.at[pl.ds(r0, ROWS_PER_TILE)])
    pltpu.sync_copy(ones_hbm, ones_v)
    plsc.subcore_barrier()

    base = wid * E_PER_TILE

    pltpu.sync_copy(src_hbm.at[pl.ds(base, CHUNK)], src_a)
    pltpu.sync_copy(dst_hbm.at[pl.ds(base, CHUNK)], dst_a)
    pltpu.async_copy(x_hbm.at[src_a], rows_a, sem_a)

    def pair_body(g, carry):
        offb = base + (2 * g + 1) * CHUNK
        offn = base + (2 * g + 2) * CHUNK
        pltpu.sync_copy(src_hbm.at[pl.ds(offb, CHUNK)], src_b)
        pltpu.sync_copy(dst_hbm.at[pl.ds(offb, CHUNK)], dst_b)
        pltpu.async_copy(x_hbm.at[src_b], rows_b, sem_b)
        pltpu.make_async_copy(x_hbm.at[src_a], rows_a, sem_a).wait()
        pltpu.sync_copy(rows_a, acc.at[dst_a], add=True)
        pltpu.sync_copy(ones_v, deg.at[dst_a], add=True)
        pltpu.sync_copy(src_hbm.at[pl.ds(offn, CHUNK)], src_a)
        pltpu.sync_copy(dst_hbm.at[pl.ds(offn, CHUNK)], dst_a)
        pltpu.async_copy(x_hbm.at[src_a], rows_a, sem_a)
        pltpu.make_async_copy(x_hbm.at[src_b], rows_b, sem_b).wait()
        pltpu.sync_copy(rows_b, acc.at[dst_b], add=True)
        pltpu.sync_copy(ones_v, deg.at[dst_b], add=True)
        return carry

    lax.fori_loop(0, N_PAIR, pair_body, 0)
    pltpu.make_async_copy(x_hbm.at[src_a], rows_a, sem_a).wait()
    pltpu.sync_copy(rows_a, acc.at[dst_a], add=True)
    pltpu.sync_copy(ones_v, deg.at[dst_a], add=True)
    plsc.subcore_barrier()

    pltpu.sync_copy(acc.at[pl.ds(r0, ROWS_PER_TILE), :],
                    part_out.at[cid, pl.ds(r0, ROWS_PER_TILE), :])
    pltpu.sync_copy(deg.at[pl.ds(r0, ROWS_PER_TILE)], dtmp)
    pltpu.sync_copy(dtmp, deg_out.at[pl.ds(cid * N_PAD + r0, ROWS_PER_TILE)])


_sc_aggregate = pl.kernel(
    _sc_body,
    out_type=(
        jax.ShapeDtypeStruct((NC, N_PAD, D_FEAT), jnp.float32),
        jax.ShapeDtypeStruct((NC * N_PAD,), jnp.float32),
    ),
    mesh=plsc.VectorSubcoreMesh(core_axis_name="c", subcore_axis_name="s",
                                num_cores=NC, num_subcores=NS),
    scratch_types=[
        pltpu.VMEM_SHARED((N_PAD, D_FEAT), jnp.float32),
        pltpu.VMEM_SHARED((N_PAD,), jnp.float32),
        pltpu.VMEM((CHUNK,), jnp.int32),
        pltpu.VMEM((CHUNK,), jnp.int32),
        pltpu.VMEM((CHUNK,), jnp.int32),
        pltpu.VMEM((CHUNK,), jnp.int32),
        pltpu.VMEM((CHUNK, D_FEAT), jnp.float32),
        pltpu.VMEM((CHUNK, D_FEAT), jnp.float32),
        pltpu.VMEM((CHUNK,), jnp.float32),
        pltpu.VMEM((ROWS_PER_TILE,), jnp.float32),
        pltpu.SemaphoreType.DMA,
        pltpu.SemaphoreType.DMA,
    ],
)


def _combine_body(p_ref, d_ref, o_ref):
    s = p_ref[0] + p_ref[1]
    d = d_ref[0] + d_ref[1]
    out = jnp.where(d > 0.0, s / d, 0.0)
    o_ref[...] = out[:N_NODES, :]


def _combine(part, degp):
    return pl.pallas_call(
        _combine_body,
        in_specs=[
            pl.BlockSpec((NC, N_PAD, D_FEAT), lambda: (0, 0, 0)),
            pl.BlockSpec((NC, N_PAD, 1), lambda: (0, 0, 0)),
        ],
        out_specs=pl.BlockSpec((N_NODES, D_FEAT), lambda: (0, 0)),
        out_shape=jax.ShapeDtypeStruct((N_NODES, D_FEAT), jnp.float32),
    )(part, degp)


@jax.jit
def kernel(input, adj):
    dst = adj[0]
    src = adj[1]
    zacc = jnp.zeros((ROWS_PER_TILE, D_FEAT), jnp.float32)
    zdeg = jnp.zeros((ROWS_PER_TILE,), jnp.float32)
    ones = jnp.ones((CHUNK,), jnp.float32)
    part, degflat = _sc_aggregate(src, dst, input, zacc, zdeg, ones)
    return _combine(part, degflat.reshape(NC, N_PAD, 1))
